# baseline (speedup 1.0000x reference)
import jax
import jax.numpy as jnp
from jax.experimental import pallas as pl
from jax.experimental.pallas import tpu as pltpu

_DIM = 4096
_BM = 1024
_NI = _DIM // _BM
_NS = 8
_NV = _DIM // 128
_NC = 4
_NBPB = _NS // 2
_NB = _NI * _NBPB


def _lhs_copy(j_hbm, lhsb, lsems, lbuf, i, v):
    return pltpu.make_async_copy(
        j_hbm.at[pl.ds(i * _BM, _BM), v, :],
        lhsb.at[lbuf, :, pl.ds(128 * v, 128)],
        lsems.at[lbuf, v])


def _rhs_copy(e_hbm, rbuf, rsems, s, c):
    return pltpu.make_async_copy(
        e_hbm.at[:, _NC * s + c, :],
        rbuf.at[:, pl.ds(128 * c, 128)],
        rsems.at[c])


def _out_copy(o_hbm, obuf, osems, i, s, c):
    return pltpu.make_async_copy(
        obuf.at[:, pl.ds(128 * c, 128)],
        o_hbm.at[pl.ds(i * _BM, _BM), _NC * s + c, :],
        osems.at[c])


def _mm_kernel(j_hbm, e_hbm, o_hbm, lhsb, rhs_a, rhs_b, out_a, out_b,
               lsems, rsems_a, rsems_b, osems_a, osems_b):
    g = pl.program_id(0)
    i = g // _NBPB
    sa = jax.lax.rem(2 * g, _NS)
    sb = sa + 1
    lbuf = jax.lax.rem(i, 2)
    ip = jnp.minimum(i + 1, _NI - 1)
    gb = jax.lax.rem(g, _NBPB)

    @pl.when(g == 0)
    def _first_loads():
        for c in range(_NC):
            _rhs_copy(e_hbm, rhs_a, rsems_a, 0, c).start()
        for v in range(_NV):
            _lhs_copy(j_hbm, lhsb, lsems, 0, 0, v).start()

    for c in range(_NC):
        _rhs_copy(e_hbm, rhs_b, rsems_b, sb, c).start()

    @pl.when(g >= 1)
    def _start_prev_out_b():
        gp = jnp.maximum(g - 1, 0)
        for c in range(_NC):
            _out_copy(o_hbm, out_b, osems_b, gp // _NBPB,
                      jax.lax.rem(2 * gp + 1, _NS), c).start()

    @pl.when(gb == 0)
    def _wait_lhs():
        for v in range(_NV):
            _lhs_copy(j_hbm, lhsb, lsems, lbuf, i, v).wait()

    _chunk = _NV // _NBPB
    for bp in range(_NBPB):
        @pl.when(jnp.logical_and(gb == bp, i + 1 < _NI))
        def _prefetch_lhs(bp=bp):
            for v in range(_chunk * bp, _chunk * (bp + 1)):
                _lhs_copy(j_hbm, lhsb, lsems, 1 - lbuf, ip, v).start()

    for c in range(_NC):
        _rhs_copy(e_hbm, rhs_a, rsems_a, sa, c).wait()

    @pl.when(g >= 1)
    def _wait_prev_out_a():
        gp = jnp.maximum(g - 1, 0)
        for c in range(_NC):
            _out_copy(o_hbm, out_a, osems_a, gp // _NBPB,
                      jax.lax.rem(2 * gp, _NS), c).wait()

    out_a[...] = jnp.dot(lhsb[lbuf], rhs_a[...],
                         preferred_element_type=jnp.float32)

    sn = jax.lax.rem(2 * g + 2, _NS)
    for c in range(_NC):
        _rhs_copy(e_hbm, rhs_a, rsems_a, sn, c).start()
    for c in range(_NC):
        _out_copy(o_hbm, out_a, osems_a, i, sa, c).start()

    for c in range(_NC):
        _rhs_copy(e_hbm, rhs_b, rsems_b, sb, c).wait()

    @pl.when(g >= 1)
    def _wait_prev_out_b():
        gp = jnp.maximum(g - 1, 0)
        for c in range(_NC):
            _out_copy(o_hbm, out_b, osems_b, gp // _NBPB,
                      jax.lax.rem(2 * gp + 1, _NS), c).wait()

    out_b[...] = jnp.dot(lhsb[lbuf], rhs_b[...],
                         preferred_element_type=jnp.float32)

    @pl.when(g == _NB - 1)
    def _drain():
        for c in range(_NC):
            _out_copy(o_hbm, out_b, osems_b, i, sb, c).start()
        for c in range(_NC):
            _rhs_copy(e_hbm, rhs_a, rsems_a, 0, c).wait()
            _out_copy(o_hbm, out_a, osems_a, i, sa, c).wait()
            _out_copy(o_hbm, out_b, osems_b, i, sb, c).wait()


def kernel(eye, jacobian):
    j3 = jacobian.reshape(_DIM, _NV, 128)
    e3 = eye.reshape(_DIM, _NV, 128)
    out = pl.pallas_call(
        _mm_kernel,
        grid=(_NB,),
        in_specs=[
            pl.BlockSpec(memory_space=pl.ANY),
            pl.BlockSpec(memory_space=pl.ANY),
        ],
        out_specs=pl.BlockSpec(memory_space=pl.ANY),
        out_shape=jax.ShapeDtypeStruct((_DIM, _NV, 128), jnp.float32),
        scratch_shapes=[
            pltpu.VMEM((2, _BM, _DIM), jnp.float32),
            pltpu.VMEM((_DIM, 128 * _NC), jnp.float32),
            pltpu.VMEM((_DIM, 128 * _NC), jnp.float32),
            pltpu.VMEM((_BM, 128 * _NC), jnp.float32),
            pltpu.VMEM((_BM, 128 * _NC), jnp.float32),
            pltpu.SemaphoreType.DMA((2, _NV)),
            pltpu.SemaphoreType.DMA((_NC,)),
            pltpu.SemaphoreType.DMA((_NC,)),
            pltpu.SemaphoreType.DMA((_NC,)),
            pltpu.SemaphoreType.DMA((_NC,)),
        ],
        compiler_params=pltpu.CompilerParams(
            dimension_semantics=("arbitrary",),
            vmem_limit_bytes=57 * 1024 * 1024,
        ),
    )(j3, e3)
    return out.reshape(_DIM * _DIM)
